# baseline (speedup 1.0000x reference)
import functools

import jax
import jax.numpy as jnp
from jax import lax
from jax.experimental import pallas as pl
from jax.experimental.pallas import tpu as pltpu
from jax.experimental.pallas import tpu_sc as plsc

VOCAB = 100000
EMBED = 64
B, S, W = 1024, 50, 20
NC, NS, L = 2, 16, 16
NWORK = NC * NS
ROWS = B * S
RPW = ROWS // NWORK
C = 16
GL = 64
G = C * W // GL
NCHUNK = RPW // C
NB = 4

_mesh = plsc.VectorSubcoreMesh(
    core_axis_name="c", subcore_axis_name="s", num_cores=NC, num_subcores=NS
)


@functools.partial(
    pl.kernel,
    out_type=jax.ShapeDtypeStruct((ROWS, EMBED), jnp.float32),
    mesh=_mesh,
    compiler_params=pltpu.CompilerParams(
        needs_layout_passes=False, use_tc_tiling_on_sc=False
    ),
    scratch_types=[
        pltpu.VMEM((RPW * W,), jnp.int32),
        pltpu.VMEM((NB, C * W, EMBED), jnp.float32),
        pltpu.VMEM((NB, C, EMBED), jnp.float32),
        pltpu.VMEM((C + L,), jnp.float32),
        pltpu.SemaphoreType.DMA,
        pltpu.SemaphoreType.DMA,
        pltpu.SemaphoreType.DMA,
        pltpu.SemaphoreType.DMA,
        pltpu.SemaphoreType.DMA,
        pltpu.SemaphoreType.DMA,
        pltpu.SemaphoreType.DMA,
        pltpu.SemaphoreType.DMA,
    ],
)
def _sc_pool(idx_hbm, table_hbm, out_hbm, idx_v, rows_v, out_v, recip_v,
             sg0, sg1, sg2, sg3, so0, so1, so2, so3):
    sem_g = (sg0, sg1, sg2, sg3)
    sem_o = (so0, so1, so2, so3)
    wid = lax.axis_index("s") * NC + lax.axis_index("c")
    lanes = lax.iota(jnp.int32, L)
    r0 = wid * RPW

    pltpu.sync_copy(idx_hbm.at[pl.ds(r0 * W, RPW * W)], idx_v)

    def fire(c, buf):
        for g in range(G):
            pltpu.async_copy(
                table_hbm.at[idx_v.at[pl.ds(c * (C * W) + g * GL, GL)]],
                rows_v.at[buf].at[pl.ds(g * GL, GL)],
                sem_g[buf],
            )

    def drain_gather(buf):
        for g in range(G):
            pltpu.make_async_copy(
                table_hbm.at[idx_v.at[pl.ds(0, GL)]],
                rows_v.at[buf].at[pl.ds(g * GL, GL)],
                sem_g[buf],
            ).wait()

    def counts(c):
        rows16 = (c * C + lanes) * W
        cnt = jnp.zeros((L,), jnp.float32)
        for w in range(W):
            ids = plsc.load_gather(idx_v, [rows16 + w])
            cnt = cnt + jnp.where(ids > 1, 1.0, 0.0).astype(jnp.float32)
        recip_v[pl.ds(0, L)] = 1.0 / (cnt + 1e-9)

    for b in range(NB - 1):
        fire(b, b)

    def quad_body(p, carry):
        for q in range(NB):
            c = NB * p + q
            nb = (q + NB - 1) % NB

            @pl.when(c + NB - 1 < NCHUNK)
            def _():
                fire(c + NB - 1, nb)

            counts(c)

            drain_gather(q)

            @pl.when(c >= NB)
            def _():
                pltpu.make_async_copy(
                    out_hbm.at[pl.ds(0, C)], out_v.at[q], sem_o[q]
                ).wait()

            def row_body(r, rc):
                scale = recip_v[pl.ds(r, L)][0]
                for k in range(EMBED // L):
                    sl = pl.ds(k * L, L)
                    t = [rows_v[q, r * W + w, sl] for w in range(W)]
                    while len(t) > 1:
                        nxt = [t[i] + t[i + 1] for i in range(0, len(t) - 1, 2)]
                        if len(t) % 2:
                            nxt.append(t[-1])
                        t = nxt
                    out_v[q, r, sl] = t[0] * scale
                return rc

            lax.fori_loop(0, C, row_body, 0, unroll=2)

            pltpu.async_copy(
                out_v.at[q], out_hbm.at[pl.ds(r0 + c * C, C)], sem_o[q]
            )
        return carry

    lax.fori_loop(0, NCHUNK // NB, quad_body, 0, unroll=False)
    for q in range(NB):
        pltpu.make_async_copy(
            out_hbm.at[pl.ds(0, C)], out_v.at[q], sem_o[q]
        ).wait()


def kernel(input, word_vectors):
    out = _sc_pool(input.reshape(-1), word_vectors)
    return out.reshape(B, S, EMBED)


# speedup vs baseline: 1.0025x; 1.0025x over previous
import functools

import jax
import jax.numpy as jnp
from jax import lax
from jax.experimental import pallas as pl
from jax.experimental.pallas import tpu as pltpu
from jax.experimental.pallas import tpu_sc as plsc

VOCAB = 100000
EMBED = 64
B, S, W = 1024, 50, 20
NC, NS, L = 2, 16, 16
NWORK = NC * NS
ROWS = B * S
RPW = ROWS // NWORK
C = 16
GL = 80
G = C * W // GL
NCHUNK = RPW // C
NB = 4

_mesh = plsc.VectorSubcoreMesh(
    core_axis_name="c", subcore_axis_name="s", num_cores=NC, num_subcores=NS
)


@functools.partial(
    pl.kernel,
    out_type=jax.ShapeDtypeStruct((ROWS, EMBED), jnp.float32),
    mesh=_mesh,
    compiler_params=pltpu.CompilerParams(
        needs_layout_passes=False, use_tc_tiling_on_sc=False
    ),
    scratch_types=[
        pltpu.VMEM((RPW * W,), jnp.int32),
        pltpu.VMEM((NB, C * W, EMBED), jnp.float32),
        pltpu.VMEM((NB, C, EMBED), jnp.float32),
        pltpu.VMEM((C + L,), jnp.float32),
        pltpu.SemaphoreType.DMA,
        pltpu.SemaphoreType.DMA,
        pltpu.SemaphoreType.DMA,
        pltpu.SemaphoreType.DMA,
        pltpu.SemaphoreType.DMA,
        pltpu.SemaphoreType.DMA,
        pltpu.SemaphoreType.DMA,
        pltpu.SemaphoreType.DMA,
    ],
)
def _sc_pool(idx_hbm, table_hbm, out_hbm, idx_v, rows_v, out_v, recip_v,
             sg0, sg1, sg2, sg3, so0, so1, so2, so3):
    sem_g = (sg0, sg1, sg2, sg3)
    sem_o = (so0, so1, so2, so3)
    wid = lax.axis_index("s") * NC + lax.axis_index("c")
    lanes = lax.iota(jnp.int32, L)
    r0 = wid * RPW

    pltpu.sync_copy(idx_hbm.at[pl.ds(r0 * W, RPW * W)], idx_v)

    def fire(c, buf):
        for g in range(G):
            pltpu.async_copy(
                table_hbm.at[idx_v.at[pl.ds(c * (C * W) + g * GL, GL)]],
                rows_v.at[buf].at[pl.ds(g * GL, GL)],
                sem_g[buf],
            )

    def drain_gather(buf):
        for g in range(G):
            pltpu.make_async_copy(
                table_hbm.at[idx_v.at[pl.ds(0, GL)]],
                rows_v.at[buf].at[pl.ds(g * GL, GL)],
                sem_g[buf],
            ).wait()

    def counts(c):
        rows16 = (c * C + lanes) * W
        cnt = jnp.zeros((L,), jnp.float32)
        for w in range(W):
            ids = plsc.load_gather(idx_v, [rows16 + w])
            cnt = cnt + jnp.where(ids > 1, 1.0, 0.0).astype(jnp.float32)
        recip_v[pl.ds(0, L)] = 1.0 / (cnt + 1e-9)

    for b in range(NB - 1):
        fire(b, b)

    def quad_body(p, carry):
        for q in range(NB):
            c = NB * p + q
            nb = (q + NB - 1) % NB

            @pl.when(c + NB - 1 < NCHUNK)
            def _():
                fire(c + NB - 1, nb)

            counts(c)

            drain_gather(q)

            @pl.when(c >= NB)
            def _():
                pltpu.make_async_copy(
                    out_hbm.at[pl.ds(0, C)], out_v.at[q], sem_o[q]
                ).wait()

            def row_body(r, rc):
                scale = recip_v[pl.ds(r, L)][0]
                for k in range(EMBED // L):
                    sl = pl.ds(k * L, L)
                    t = [rows_v[q, r * W + w, sl] for w in range(W)]
                    while len(t) > 1:
                        nxt = [t[i] + t[i + 1] for i in range(0, len(t) - 1, 2)]
                        if len(t) % 2:
                            nxt.append(t[-1])
                        t = nxt
                    out_v[q, r, sl] = t[0] * scale
                return rc

            lax.fori_loop(0, C, row_body, 0, unroll=2)

            pltpu.async_copy(
                out_v.at[q], out_hbm.at[pl.ds(r0 + c * C, C)], sem_o[q]
            )
        return carry

    lax.fori_loop(0, NCHUNK // NB, quad_body, 0, unroll=False)
    for q in range(NB):
        pltpu.make_async_copy(
            out_hbm.at[pl.ds(0, C)], out_v.at[q], sem_o[q]
        ).wait()


def kernel(input, word_vectors):
    out = _sc_pool(input.reshape(-1), word_vectors)
    return out.reshape(B, S, EMBED)
